# Initial kernel scaffold; baseline (speedup 1.0000x reference)
import jax
import jax.numpy as jnp
from jax.experimental import pallas as pl


def kernel(x, router_mask, in_proj_w, in_proj_b, out_proj_w, out_proj_b, ln1_g, ln1_b, ln2_g, ln2_b, gate_w, gate_b, expert_w, expert_b):
    raise NotImplementedError("write your pallas kernel here")


# speedup vs baseline: 1.8225x; 1.8225x over previous
import jax
import jax.numpy as jnp
from jax.experimental import pallas as pl
from jax.experimental.pallas import tpu as pltpu

S, D, H, E = 2048, 768, 8, 64
DH = D // H
QB = 512
NQB = S // QB


def _attn_kernel(x_ref, wq_ref, wk_ref, wv_ref, bq_ref, bk_ref, bv_ref,
                 wo_ref, bo_ref, ln1g_ref, ln1b_ref, gw_ref, gb_ref,
                 x1_ref, topv_ref, topi_ref, acc_ref):
    h = pl.program_id(0)

    @pl.when(h == 0)
    def _init():
        acc_ref[...] = jnp.zeros_like(acc_ref)

    x = x_ref[...]
    wk = wk_ref[0]
    wv = wv_ref[0]
    k = jax.lax.dot_general(x, wk, (((1,), (1,)), ((), ()))) + bk_ref[0]
    v = jax.lax.dot_general(x, wv, (((1,), (1,)), ((), ()))) + bv_ref[0]
    wq = wq_ref[0]
    wo = wo_ref[0]

    def qblock(i, carry):
        xq = x_ref[pl.ds(i * QB, QB), :]
        q = jax.lax.dot_general(xq, wq, (((1,), (1,)), ((), ()))) + bq_ref[0]
        s = jax.lax.dot_general(q, k, (((1,), (1,)), ((), ()))) \
            / jnp.sqrt(jnp.float32(DH))
        C = S // 2
        m = jnp.full((QB, 1), -jnp.inf, jnp.float32)
        l = jnp.zeros((QB, 1), jnp.float32)
        acc = jnp.zeros((QB, DH), jnp.float32)
        for c in range(2):
            sc = s[:, c * C:(c + 1) * C]
            mc = jnp.max(sc, axis=1, keepdims=True)
            mn = jnp.maximum(m, mc)
            corr = jnp.exp(m - mn)
            u = jnp.exp(sc - mn)
            l = l * corr + jnp.sum(u, axis=1, keepdims=True)
            pv = jax.lax.dot_general(u, v[c * C:(c + 1) * C, :],
                                     (((1,), (0,)), ((), ())))
            acc = corr * acc + pv
            m = mn
        o = acc / l
        contrib = jax.lax.dot_general(o, wo, (((1,), (1,)), ((), ())))
        acc_ref[pl.ds(i * QB, QB), :] += contrib
        return carry

    jax.lax.fori_loop(0, NQB, qblock, 0)

    @pl.when(h == H - 1)
    def _tail():
        y = x_ref[...] + (acc_ref[...] + bo_ref[...])
        mu = jnp.mean(y, axis=1, keepdims=True)
        var = jnp.mean((y - mu) ** 2, axis=1, keepdims=True)
        x1 = (y - mu) / jnp.sqrt(var + 1e-5) * ln1g_ref[...] + ln1b_ref[...]
        x1_ref[...] = x1
        g = jax.lax.dot_general(x1, gw_ref[...], (((1,), (1,)), ((), ()))) \
            + gb_ref[...]
        topv = jnp.max(g, axis=1, keepdims=True)
        ei = jax.lax.broadcasted_iota(jnp.int32, (S, E), 1)
        topi = jnp.min(jnp.where(g >= topv, ei, E), axis=1, keepdims=True)
        topv_ref[...] = topv
        topi_ref[...] = topi


def _moe_kernel(x1_ref, topv_ref, topi_ref, w_ref, b_ref, ln2g_ref, ln2b_ref,
                x2_ref, acc_ref):
    e = pl.program_id(0)

    @pl.when(e == 0)
    def _init():
        acc_ref[...] = jnp.zeros_like(acc_ref)

    w = w_ref[0]

    def sblock(i, carry):
        x1 = x1_ref[pl.ds(i * QB, QB), :]
        y = jax.lax.dot_general(x1, w, (((1,), (1,)), ((), ()))) + b_ref[0]
        cmb = jnp.where(topi_ref[pl.ds(i * QB, QB), :] == e,
                        topv_ref[pl.ds(i * QB, QB), :], 0.0)
        acc_ref[pl.ds(i * QB, QB), :] += cmb * y
        return carry

    jax.lax.fori_loop(0, NQB, sblock, 0)

    @pl.when(e == E - 1)
    def _tail():
        z = x1_ref[...] + acc_ref[...]
        mu = jnp.mean(z, axis=1, keepdims=True)
        var = jnp.mean((z - mu) ** 2, axis=1, keepdims=True)
        x2_ref[...] = (z - mu) / jnp.sqrt(var + 1e-5) \
            * ln2g_ref[...] + ln2b_ref[...]


def _const2(shape):
    return pl.BlockSpec(shape, lambda *_: tuple(0 for _ in shape))


def kernel(x, router_mask, in_proj_w, in_proj_b, out_proj_w, out_proj_b,
           ln1_g, ln1_b, ln2_g, ln2_b, gate_w, gate_b, expert_w, expert_b):
    del router_mask
    x2d = x.reshape(S, D)
    wq = in_proj_w[0 * D:1 * D].reshape(H, DH, D)
    wk = in_proj_w[1 * D:2 * D].reshape(H, DH, D)
    wv = in_proj_w[2 * D:3 * D].reshape(H, DH, D)
    bq = in_proj_b[0 * D:1 * D].reshape(H, 1, DH)
    bk = in_proj_b[1 * D:2 * D].reshape(H, 1, DH)
    bv = in_proj_b[2 * D:3 * D].reshape(H, 1, DH)
    wo = out_proj_w.reshape(D, H, DH).transpose(1, 0, 2)

    head_spec_w = pl.BlockSpec((1, DH, D), lambda h: (h, 0, 0))
    head_spec_b = pl.BlockSpec((1, 1, DH), lambda h: (h, 0, 0))
    x1, topv, topi = pl.pallas_call(
        _attn_kernel,
        grid=(H,),
        in_specs=[
            _const2((S, D)),
            head_spec_w, head_spec_w, head_spec_w,
            head_spec_b, head_spec_b, head_spec_b,
            pl.BlockSpec((1, D, DH), lambda h: (h, 0, 0)),
            _const2((1, D)), _const2((1, D)), _const2((1, D)),
            _const2((E, D)), _const2((1, E)),
        ],
        out_specs=[_const2((S, D)), _const2((S, 1)), _const2((S, 1))],
        out_shape=[
            jax.ShapeDtypeStruct((S, D), jnp.float32),
            jax.ShapeDtypeStruct((S, 1), jnp.float32),
            jax.ShapeDtypeStruct((S, 1), jnp.int32),
        ],
        scratch_shapes=[pltpu.VMEM((S, D), jnp.float32)],
    )(x2d, wq, wk, wv, bq, bk, bv, wo,
      out_proj_b.reshape(1, D), ln1_g.reshape(1, D), ln1_b.reshape(1, D),
      gate_w, gate_b.reshape(1, E))

    x2 = pl.pallas_call(
        _moe_kernel,
        grid=(E,),
        in_specs=[
            _const2((S, D)), _const2((S, 1)), _const2((S, 1)),
            pl.BlockSpec((1, D, D), lambda e: (e, 0, 0)),
            pl.BlockSpec((1, 1, D), lambda e: (e, 0, 0)),
            _const2((1, D)), _const2((1, D)),
        ],
        out_specs=_const2((S, D)),
        out_shape=jax.ShapeDtypeStruct((S, D), jnp.float32),
        scratch_shapes=[pltpu.VMEM((S, D), jnp.float32)],
    )(x1, topv, topi, expert_w, expert_b.reshape(E, 1, D), ln2_g.reshape(1, D),
      ln2_b.reshape(1, D))

    return x2.reshape(S, 1, D)


# speedup vs baseline: 2.7267x; 1.4961x over previous
import functools

import jax
import jax.numpy as jnp
from jax import lax
from jax.experimental import pallas as pl
from jax.experimental.pallas import tpu as pltpu
from jax.experimental.pallas import tpu_sc as plsc

S, D, H, E = 2048, 768, 8, 64
DH = D // H
QB = 512
NQB = S // QB
TB = 128
NBLK = S // TB + E
SP = NBLK * TB
NW = 32
CHUNK = S // NW


def _attn_kernel(x_ref, wq_ref, wk_ref, wv_ref, bq_ref, bk_ref, bv_ref,
                 wo_ref, bo_ref, ln1g_ref, ln1b_ref, gw_ref, gb_ref,
                 x1_ref, topv_ref, dest_ref, be_ref, acc_ref):
    h = pl.program_id(0)

    @pl.when(h == 0)
    def _init():
        acc_ref[...] = jnp.zeros_like(acc_ref)

    x = x_ref[...]
    wk = wk_ref[0]
    wv = wv_ref[0]
    k = jax.lax.dot_general(x, wk, (((1,), (1,)), ((), ()))) + bk_ref[0]
    v = jax.lax.dot_general(x, wv, (((1,), (1,)), ((), ()))) + bv_ref[0]
    wq = wq_ref[0]
    wo = wo_ref[0]

    def qblock(i, carry):
        xq = x_ref[pl.ds(i * QB, QB), :]
        q = jax.lax.dot_general(xq, wq, (((1,), (1,)), ((), ()))) + bq_ref[0]
        s = jax.lax.dot_general(q, k, (((1,), (1,)), ((), ()))) \
            / jnp.sqrt(jnp.float32(DH))
        C = S // 2
        m = jnp.full((QB, 1), -jnp.inf, jnp.float32)
        l = jnp.zeros((QB, 1), jnp.float32)
        acc = jnp.zeros((QB, DH), jnp.float32)
        for c in range(2):
            sc = s[:, c * C:(c + 1) * C]
            mc = jnp.max(sc, axis=1, keepdims=True)
            mn = jnp.maximum(m, mc)
            corr = jnp.exp(m - mn)
            u = jnp.exp(sc - mn)
            l = l * corr + jnp.sum(u, axis=1, keepdims=True)
            pv = jax.lax.dot_general(u, v[c * C:(c + 1) * C, :],
                                     (((1,), (0,)), ((), ())))
            acc = corr * acc + pv
            m = mn
        o = acc / l
        contrib = jax.lax.dot_general(o, wo, (((1,), (1,)), ((), ())))
        acc_ref[pl.ds(i * QB, QB), :] += contrib
        return carry

    jax.lax.fori_loop(0, NQB, qblock, 0)

    @pl.when(h == H - 1)
    def _tail():
        y = x_ref[...] + (acc_ref[...] + bo_ref[...])
        mu = jnp.mean(y, axis=1, keepdims=True)
        var = jnp.mean((y - mu) ** 2, axis=1, keepdims=True)
        x1 = (y - mu) / jnp.sqrt(var + 1e-5) * ln1g_ref[...] + ln1b_ref[...]
        x1_ref[...] = x1
        g = jax.lax.dot_general(x1, gw_ref[...], (((1,), (1,)), ((), ()))) \
            + gb_ref[...]
        topv = jnp.max(g, axis=1, keepdims=True)
        ei = jax.lax.broadcasted_iota(jnp.int32, (S, E), 1).astype(jnp.float32)
        topif = jnp.min(jnp.where(g >= topv, ei, jnp.float32(E)),
                        axis=1, keepdims=True)
        topv_ref[...] = topv

        onehot = (topif == ei).astype(jnp.float32)
        ltri = (jax.lax.broadcasted_iota(jnp.int32, (S, S), 0) >=
                jax.lax.broadcasted_iota(jnp.int32, (S, S), 1)
                ).astype(jnp.float32)
        incl = jax.lax.dot_general(
            ltri, onehot, (((1,), (0,)), ((), ())))
        counts = incl[S - 1:S, :]
        ee_r = jax.lax.broadcasted_iota(jnp.int32, (E, E), 0)
        ee_c = jax.lax.broadcasted_iota(jnp.int32, (E, E), 1)
        u_strict = (ee_r < ee_c).astype(jnp.float32)
        u_incl = (ee_r <= ee_c).astype(jnp.float32)
        nb = jnp.floor((counts + jnp.float32(TB - 1))
                       * jnp.float32(1.0 / TB))
        cnb_incl = jax.lax.dot_general(
            nb, u_incl, (((1,), (0,)), ((), ())))
        cnb_excl = cnb_incl - nb
        offs_pad = cnb_excl * jnp.float32(TB)
        dest = jnp.sum(onehot * (offs_pad + incl - 1.0), axis=1,
                       keepdims=True)
        dest_ref[...] = dest.astype(jnp.int32)
        bi = jax.lax.broadcasted_iota(jnp.int32, (NBLK, 1), 0) \
            .astype(jnp.float32)
        be = jnp.sum((bi >= cnb_incl).astype(jnp.float32), axis=1,
                     keepdims=True)
        be_ref[...] = be.astype(jnp.int32)


def _group_kernel(be_ref, xs_ref, w_ref, b_ref, ys_ref):
    i = pl.program_id(0)

    @pl.when(be_ref[i] < E)
    def _():
        y = jax.lax.dot_general(xs_ref[...], w_ref[0],
                                (((1,), (1,)), ((), ()))) + b_ref[0]
        ys_ref[...] = y


def _ln2_kernel(x1_ref, topv_ref, moe_ref, ln2g_ref, ln2b_ref, x2_ref):
    z = x1_ref[...] + topv_ref[...] * moe_ref[...]
    mu = jnp.mean(z, axis=1, keepdims=True)
    var = jnp.mean((z - mu) ** 2, axis=1, keepdims=True)
    x2_ref[...] = (z - mu) / jnp.sqrt(var + 1e-5) \
        * ln2g_ref[...] + ln2b_ref[...]


def _sc_scatter(x1, dest):
    mesh = plsc.VectorSubcoreMesh(core_axis_name="c", subcore_axis_name="s")

    @functools.partial(
        pl.kernel, mesh=mesh,
        out_type=jax.ShapeDtypeStruct((SP, D), jnp.float32),
        scratch_types=[
            pltpu.VMEM((CHUNK,), jnp.int32),
            pltpu.VMEM((CHUNK, D), jnp.float32),
            pltpu.SemaphoreType.DMA,
        ],
    )
    def k(x1_hbm, dest_hbm, out_hbm, idx_v, rows_v, sem):
        wid = lax.axis_index("s") * 2 + lax.axis_index("c")
        base = wid * CHUNK
        pltpu.sync_copy(dest_hbm.at[pl.ds(base, CHUNK)], idx_v)
        pltpu.sync_copy(x1_hbm.at[pl.ds(base, CHUNK)], rows_v)
        pltpu.async_copy(rows_v, out_hbm.at[idx_v], sem).wait()

    return k(x1, dest)


def _sc_gather(ys, dest):
    mesh = plsc.VectorSubcoreMesh(core_axis_name="c", subcore_axis_name="s")

    @functools.partial(
        pl.kernel, mesh=mesh,
        out_type=jax.ShapeDtypeStruct((S, D), jnp.float32),
        scratch_types=[
            pltpu.VMEM((CHUNK,), jnp.int32),
            pltpu.VMEM((CHUNK, D), jnp.float32),
            pltpu.SemaphoreType.DMA,
        ],
    )
    def k(ys_hbm, dest_hbm, out_hbm, idx_v, rows_v, sem):
        wid = lax.axis_index("s") * 2 + lax.axis_index("c")
        base = wid * CHUNK
        pltpu.sync_copy(dest_hbm.at[pl.ds(base, CHUNK)], idx_v)
        pltpu.async_copy(ys_hbm.at[idx_v], rows_v, sem).wait()
        pltpu.sync_copy(rows_v, out_hbm.at[pl.ds(base, CHUNK)])

    return k(ys, dest)


def _const2(shape):
    return pl.BlockSpec(shape, lambda *_: tuple(0 for _ in shape))


def kernel(x, router_mask, in_proj_w, in_proj_b, out_proj_w, out_proj_b,
           ln1_g, ln1_b, ln2_g, ln2_b, gate_w, gate_b, expert_w, expert_b):
    del router_mask
    x2d = x.reshape(S, D)
    wq = in_proj_w[0 * D:1 * D].reshape(H, DH, D)
    wk = in_proj_w[1 * D:2 * D].reshape(H, DH, D)
    wv = in_proj_w[2 * D:3 * D].reshape(H, DH, D)
    bq = in_proj_b[0 * D:1 * D].reshape(H, 1, DH)
    bk = in_proj_b[1 * D:2 * D].reshape(H, 1, DH)
    bv = in_proj_b[2 * D:3 * D].reshape(H, 1, DH)
    wo = out_proj_w.reshape(D, H, DH).transpose(1, 0, 2)

    head_spec_w = pl.BlockSpec((1, DH, D), lambda h: (h, 0, 0))
    head_spec_b = pl.BlockSpec((1, 1, DH), lambda h: (h, 0, 0))
    x1, topv, dest, be = pl.pallas_call(
        _attn_kernel,
        grid=(H,),
        in_specs=[
            _const2((S, D)),
            head_spec_w, head_spec_w, head_spec_w,
            head_spec_b, head_spec_b, head_spec_b,
            pl.BlockSpec((1, D, DH), lambda h: (h, 0, 0)),
            _const2((1, D)), _const2((1, D)), _const2((1, D)),
            _const2((E, D)), _const2((1, E)),
        ],
        out_specs=[_const2((S, D)), _const2((S, 1)), _const2((S, 1)),
                   _const2((NBLK, 1))],
        out_shape=[
            jax.ShapeDtypeStruct((S, D), jnp.float32),
            jax.ShapeDtypeStruct((S, 1), jnp.float32),
            jax.ShapeDtypeStruct((S, 1), jnp.int32),
            jax.ShapeDtypeStruct((NBLK, 1), jnp.int32),
        ],
        scratch_shapes=[pltpu.VMEM((S, D), jnp.float32)],
    )(x2d, wq, wk, wv, bq, bk, bv, wo,
      out_proj_b.reshape(1, D), ln1_g.reshape(1, D), ln1_b.reshape(1, D),
      gate_w, gate_b.reshape(1, E))

    dest1 = dest.reshape(S)
    xs = _sc_scatter(x1, dest1)

    grid_spec = pltpu.PrefetchScalarGridSpec(
        num_scalar_prefetch=1,
        grid=(NBLK,),
        in_specs=[
            pl.BlockSpec((TB, D), lambda i, be_: (i, 0)),
            pl.BlockSpec((1, D, D),
                         lambda i, be_: (jnp.minimum(be_[i], E - 1), 0, 0)),
            pl.BlockSpec((1, 1, D),
                         lambda i, be_: (jnp.minimum(be_[i], E - 1), 0, 0)),
        ],
        out_specs=pl.BlockSpec((TB, D), lambda i, be_: (i, 0)),
    )
    ys = pl.pallas_call(
        _group_kernel,
        grid_spec=grid_spec,
        out_shape=jax.ShapeDtypeStruct((SP, D), jnp.float32),
    )(be.reshape(NBLK), xs, expert_w, expert_b.reshape(E, 1, D))

    moe = _sc_gather(ys, dest1)

    x2 = pl.pallas_call(
        _ln2_kernel,
        in_specs=[_const2((S, D)), _const2((S, 1)), _const2((S, D)),
                  _const2((1, D)), _const2((1, D))],
        out_specs=_const2((S, D)),
        out_shape=jax.ShapeDtypeStruct((S, D), jnp.float32),
    )(x1, topv, moe, ln2_g.reshape(1, D), ln2_b.reshape(1, D))

    return x2.reshape(S, 1, D)


# speedup vs baseline: 2.8112x; 1.0310x over previous
import functools

import jax
import jax.numpy as jnp
from jax import lax
from jax.experimental import pallas as pl
from jax.experimental.pallas import tpu as pltpu
from jax.experimental.pallas import tpu_sc as plsc

S, D, H, E = 2048, 768, 8, 64
DH = D // H
QB = 512
NQB = S // QB
TB = 128
NBLK = S // TB + E
SP = NBLK * TB
NW = 32
CHUNK = S // NW


def _attn_kernel(x_ref, wq_ref, wk_ref, wv_ref, bq_ref, bk_ref, bv_ref,
                 wo_ref, bo_ref, ln1g_ref, ln1b_ref, gw_ref, gb_ref,
                 x1_ref, topv_ref, dest_ref, be_ref, acc_ref):
    h = pl.program_id(0)

    @pl.when(h == 0)
    def _init():
        acc_ref[...] = jnp.zeros_like(acc_ref)

    x = x_ref[...]
    wk = wk_ref[0]
    wv = wv_ref[0]
    k = jax.lax.dot_general(x, wk, (((1,), (1,)), ((), ()))) + bk_ref[0]
    v = jax.lax.dot_general(x, wv, (((1,), (1,)), ((), ()))) + bv_ref[0]
    wq = wq_ref[0]
    wo = wo_ref[0]

    def qblock(i):
        xq = x_ref[pl.ds(i * QB, QB), :]
        q = jax.lax.dot_general(xq, wq, (((1,), (1,)), ((), ()))) + bq_ref[0]
        s = jax.lax.dot_general(q, k, (((1,), (1,)), ((), ()))) \
            / jnp.sqrt(jnp.float32(DH))
        C = S // 2
        m = jnp.full((QB, 1), -jnp.inf, jnp.float32)
        l = jnp.zeros((QB, 1), jnp.float32)
        acc = jnp.zeros((QB, DH), jnp.float32)
        for c in range(2):
            sc = s[:, c * C:(c + 1) * C]
            mc = jnp.max(sc, axis=1, keepdims=True)
            mn = jnp.maximum(m, mc)
            corr = jnp.exp(m - mn)
            u = jnp.exp(sc - mn)
            l = l * corr + jnp.sum(u, axis=1, keepdims=True)
            pv = jax.lax.dot_general(u, v[c * C:(c + 1) * C, :],
                                     (((1,), (0,)), ((), ())))
            acc = corr * acc + pv
            m = mn
        o = acc / l
        contrib = jax.lax.dot_general(o, wo, (((1,), (1,)), ((), ())))
        acc_ref[pl.ds(i * QB, QB), :] += contrib

    for i in range(NQB):
        qblock(i)

    @pl.when(h == H - 1)
    def _tail():
        y = x_ref[...] + (acc_ref[...] + bo_ref[...])
        inv_d = jnp.float32(1.0 / D)
        mu = jnp.sum(y, axis=1, keepdims=True) * inv_d
        var = jnp.sum((y - mu) ** 2, axis=1, keepdims=True) * inv_d
        x1 = (y - mu) / jnp.sqrt(var + 1e-5) * ln1g_ref[...] + ln1b_ref[...]
        x1_ref[...] = x1
        g = jax.lax.dot_general(x1, gw_ref[...], (((1,), (1,)), ((), ()))) \
            + gb_ref[...]
        topv = jnp.max(g, axis=1, keepdims=True)
        ei = jax.lax.broadcasted_iota(jnp.int32, (S, E), 1).astype(jnp.float32)
        topif = jnp.min(jnp.where(g >= topv, ei, jnp.float32(E)),
                        axis=1, keepdims=True)
        topv_ref[...] = topv

        onehot = (topif == ei).astype(jnp.float32)
        ltri = (jax.lax.broadcasted_iota(jnp.int32, (S, S), 0) >=
                jax.lax.broadcasted_iota(jnp.int32, (S, S), 1)
                ).astype(jnp.float32)
        incl = jax.lax.dot_general(
            ltri, onehot, (((1,), (0,)), ((), ())))
        counts = incl[S - 1:S, :]
        ee_r = jax.lax.broadcasted_iota(jnp.int32, (E, E), 0)
        ee_c = jax.lax.broadcasted_iota(jnp.int32, (E, E), 1)
        u_strict = (ee_r < ee_c).astype(jnp.float32)
        u_incl = (ee_r <= ee_c).astype(jnp.float32)
        nb = jnp.floor((counts + jnp.float32(TB - 1))
                       * jnp.float32(1.0 / TB))
        cnb_incl = jax.lax.dot_general(
            nb, u_incl, (((1,), (0,)), ((), ())))
        cnb_excl = cnb_incl - nb
        offs_pad = cnb_excl * jnp.float32(TB)
        dest = jnp.sum(onehot * (offs_pad + incl - 1.0), axis=1,
                       keepdims=True)
        dest_ref[...] = dest.astype(jnp.int32)
        bi = jax.lax.broadcasted_iota(jnp.int32, (NBLK, 1), 0) \
            .astype(jnp.float32)
        be = jnp.sum((bi >= cnb_incl).astype(jnp.float32), axis=1,
                     keepdims=True)
        be_ref[...] = be.astype(jnp.int32)


def _group_kernel(be_ref, xs_ref, w_ref, b_ref, ys_ref):
    i = pl.program_id(0)

    @pl.when(be_ref[i] < E)
    def _():
        y = jax.lax.dot_general(xs_ref[...], w_ref[0],
                                (((1,), (1,)), ((), ()))) + b_ref[0]
        ys_ref[...] = y


def _ln2_kernel(x1_ref, topv_ref, moe_ref, ln2g_ref, ln2b_ref, x2_ref):
    z = x1_ref[...] + topv_ref[...] * moe_ref[...]
    inv_d = jnp.float32(1.0 / D)
    mu = jnp.sum(z, axis=1, keepdims=True) * inv_d
    var = jnp.sum((z - mu) ** 2, axis=1, keepdims=True) * inv_d
    x2_ref[...] = (z - mu) / jnp.sqrt(var + 1e-5) \
        * ln2g_ref[...] + ln2b_ref[...]


def _sc_scatter(x1, dest):
    mesh = plsc.VectorSubcoreMesh(core_axis_name="c", subcore_axis_name="s")

    @functools.partial(
        pl.kernel, mesh=mesh,
        out_type=jax.ShapeDtypeStruct((SP, D), jnp.float32),
        scratch_types=[
            pltpu.VMEM((CHUNK,), jnp.int32),
            pltpu.VMEM((CHUNK, D), jnp.float32),
            pltpu.SemaphoreType.DMA,
        ],
    )
    def k(x1_hbm, dest_hbm, out_hbm, idx_v, rows_v, sem):
        wid = lax.axis_index("s") * 2 + lax.axis_index("c")
        base = wid * CHUNK
        pltpu.sync_copy(dest_hbm.at[pl.ds(base, CHUNK)], idx_v)
        pltpu.sync_copy(x1_hbm.at[pl.ds(base, CHUNK)], rows_v)
        pltpu.async_copy(rows_v, out_hbm.at[idx_v], sem).wait()

    return k(x1, dest)


def _sc_gather(ys, dest):
    mesh = plsc.VectorSubcoreMesh(core_axis_name="c", subcore_axis_name="s")

    @functools.partial(
        pl.kernel, mesh=mesh,
        out_type=jax.ShapeDtypeStruct((S, D), jnp.float32),
        scratch_types=[
            pltpu.VMEM((CHUNK,), jnp.int32),
            pltpu.VMEM((CHUNK, D), jnp.float32),
            pltpu.SemaphoreType.DMA,
        ],
    )
    def k(ys_hbm, dest_hbm, out_hbm, idx_v, rows_v, sem):
        wid = lax.axis_index("s") * 2 + lax.axis_index("c")
        base = wid * CHUNK
        pltpu.sync_copy(dest_hbm.at[pl.ds(base, CHUNK)], idx_v)
        pltpu.async_copy(ys_hbm.at[idx_v], rows_v, sem).wait()
        pltpu.sync_copy(rows_v, out_hbm.at[pl.ds(base, CHUNK)])

    return k(ys, dest)


def _const2(shape):
    return pl.BlockSpec(shape, lambda *_: tuple(0 for _ in shape))


def kernel(x, router_mask, in_proj_w, in_proj_b, out_proj_w, out_proj_b,
           ln1_g, ln1_b, ln2_g, ln2_b, gate_w, gate_b, expert_w, expert_b):
    del router_mask
    x2d = x.reshape(S, D)
    wq = in_proj_w[0 * D:1 * D].reshape(H, DH, D)
    wk = in_proj_w[1 * D:2 * D].reshape(H, DH, D)
    wv = in_proj_w[2 * D:3 * D].reshape(H, DH, D)
    bq = in_proj_b[0 * D:1 * D].reshape(H, 1, DH)
    bk = in_proj_b[1 * D:2 * D].reshape(H, 1, DH)
    bv = in_proj_b[2 * D:3 * D].reshape(H, 1, DH)
    wo = out_proj_w.reshape(D, H, DH).transpose(1, 0, 2)

    head_spec_w = pl.BlockSpec((1, DH, D), lambda h: (h, 0, 0))
    head_spec_b = pl.BlockSpec((1, 1, DH), lambda h: (h, 0, 0))
    x1, topv, dest, be = pl.pallas_call(
        _attn_kernel,
        grid=(H,),
        in_specs=[
            _const2((S, D)),
            head_spec_w, head_spec_w, head_spec_w,
            head_spec_b, head_spec_b, head_spec_b,
            pl.BlockSpec((1, D, DH), lambda h: (h, 0, 0)),
            _const2((1, D)), _const2((1, D)), _const2((1, D)),
            _const2((E, D)), _const2((1, E)),
        ],
        out_specs=[_const2((S, D)), _const2((S, 1)), _const2((S, 1)),
                   _const2((NBLK, 1))],
        out_shape=[
            jax.ShapeDtypeStruct((S, D), jnp.float32),
            jax.ShapeDtypeStruct((S, 1), jnp.float32),
            jax.ShapeDtypeStruct((S, 1), jnp.int32),
            jax.ShapeDtypeStruct((NBLK, 1), jnp.int32),
        ],
        scratch_shapes=[pltpu.VMEM((S, D), jnp.float32)],
    )(x2d, wq, wk, wv, bq, bk, bv, wo,
      out_proj_b.reshape(1, D), ln1_g.reshape(1, D), ln1_b.reshape(1, D),
      gate_w, gate_b.reshape(1, E))

    dest1 = dest.reshape(S)
    xs = _sc_scatter(x1, dest1)

    grid_spec = pltpu.PrefetchScalarGridSpec(
        num_scalar_prefetch=1,
        grid=(NBLK,),
        in_specs=[
            pl.BlockSpec((TB, D), lambda i, be_: (i, 0)),
            pl.BlockSpec((1, D, D),
                         lambda i, be_: (jnp.minimum(be_[i], E - 1), 0, 0)),
            pl.BlockSpec((1, 1, D),
                         lambda i, be_: (jnp.minimum(be_[i], E - 1), 0, 0)),
        ],
        out_specs=pl.BlockSpec((TB, D), lambda i, be_: (i, 0)),
    )
    ys = pl.pallas_call(
        _group_kernel,
        grid_spec=grid_spec,
        out_shape=jax.ShapeDtypeStruct((SP, D), jnp.float32),
    )(be.reshape(NBLK), xs, expert_w, expert_b.reshape(E, 1, D))

    moe = _sc_gather(ys, dest1)

    x2 = pl.pallas_call(
        _ln2_kernel,
        in_specs=[_const2((S, D)), _const2((S, 1)), _const2((S, D)),
                  _const2((1, D)), _const2((1, D))],
        out_specs=_const2((S, D)),
        out_shape=jax.ShapeDtypeStruct((S, D), jnp.float32),
    )(x1, topv, moe, ln2_g.reshape(1, D), ln2_b.reshape(1, D))

    return x2.reshape(S, 1, D)
